# baseline (speedup 1.0000x reference)
import functools

import jax
import jax.numpy as jnp
from jax.experimental import pallas as pl
from jax.experimental.pallas import tpu as pltpu

N_SAMPLES = 512
N_AGENTS = 32
D = 128
EDIM = 2
N = N_SAMPLES * N_AGENTS

SB = 64
BN = SB * N_AGENTS
GRID = N_SAMPLES // SB
CH = 4
CHN = CH * N_AGENTS
NCHUNK = SB // CH

_LN2 = 0.6931471805599453


def _pq(xb, cb, wd, ws, aux, w_e_rows, b_row):
    ce = (cb[:, 0:1] * aux[w_e_rows[0]:w_e_rows[0] + 1, :]
          + cb[:, 1:2] * aux[w_e_rows[1]:w_e_rows[1] + 1, :])
    dot = functools.partial(jax.lax.dot_general,
                            dimension_numbers=(((1,), (0,)), ((), ())),
                            preferred_element_type=jnp.float32,
                            precision=jax.lax.Precision.HIGHEST)
    p = dot(xb, wd) + ce + aux[b_row:b_row + 1, :]
    q = dot(xb, ws) - ce
    return p, q


def _msg_scaled(v, s2):
    t = jnp.tanh(v)
    m = jnp.log(1.0 + jnp.exp(s2))
    return m + t * m


def _pairwise_chunk(pf, qf, ps, qs):
    pf3 = pf.reshape(CH, N_AGENTS, D)
    qf3 = qf.reshape(CH, N_AGENTS, D)
    ps3 = ps.reshape(CH, N_AGENTS, D)
    qs3 = qs.reshape(CH, N_AGENTS, D)
    acc = -_msg_scaled(pf + qf, ps + qs).reshape(CH, N_AGENTS, D)
    for i in range(N_AGENTS):
        acc = acc + _msg_scaled(pf3 + qf3[:, i:i + 1, :],
                                ps3 + qs3[:, i:i + 1, :])
    return acc.reshape(CHN, D) * 0.5


def _pairwise_to_scratch(pf, qf, ps, qs, agg_scr, i, st_ref):
    ssum = jnp.zeros((1, D), jnp.float32)
    ssq = jnp.zeros((1, D), jnp.float32)
    for c in range(NCHUNK):
        sl = slice(c * CHN, (c + 1) * CHN)
        aggc = _pairwise_chunk(pf[sl], qf[sl], ps[sl], qs[sl])
        agg_scr[pl.ds(i * BN + c * CHN, CHN), :] = aggc
        ssum = ssum + aggc.sum(axis=0, keepdims=True)
        ssq = ssq + (aggc * aggc).sum(axis=0, keepdims=True)
    _accum_finalize(st_ref, ssum, ssq, i)


def _accum_finalize(st_ref, ssum, ssq, i):
    @pl.when(i == 0)
    def _():
        st_ref[:, :] = jnp.zeros_like(st_ref)
    st_ref[0:1, :] = st_ref[0:1, :] + ssum
    st_ref[1:2, :] = st_ref[1:2, :] + ssq

    @pl.when(i == GRID - 1)
    def _():
        mu = st_ref[0:1, :] * (1.0 / N)
        var = st_ref[1:2, :] * (1.0 / N) - mu * mu
        st_ref[0:1, :] = mu
        st_ref[1:2, :] = jax.lax.rsqrt(var + 1e-5)


def _bn_res_relu(xb, agg, st_ref, aux):
    aggn = (agg - st_ref[0:1, :]) * st_ref[1:2, :] * aux[2:3, :] + aux[3:4, :]
    return jax.nn.relu(xb + aggn)


def _kall(x_ref, c_ref, wfd1, wfs1, wsd1, wss1, aux1,
          wfd2, wfs2, wsd2, wss2, aux2, out_ref,
          agg_scr, x1_scr, st1_scr, st2_scr):
    p = pl.program_id(0)
    i = pl.program_id(1)
    rows = pl.ds(i * BN, BN)

    @pl.when(p == 0)
    def _phase0():
        pf, qf = _pq(x_ref[:, :], c_ref[:, :], wfd1[:, :], wfs1[:, :],
                     aux1[:, :], (4, 5), 0)
        ps, qs = _pq(x_ref[:, :], c_ref[:, :], wsd1[:, :], wss1[:, :],
                     aux1[:, :], (6, 7), 1)
        _pairwise_to_scratch(pf, qf, ps, qs, agg_scr, i, st1_scr)

    @pl.when(p == 1)
    def _phase1():
        x1 = _bn_res_relu(x_ref[:, :], agg_scr[rows, :], st1_scr, aux1[:, :])
        x1_scr[rows, :] = x1
        pf, qf = _pq(x1, c_ref[:, :], wfd2[:, :], wfs2[:, :],
                     aux2[:, :], (4, 5), 0)
        ps, qs = _pq(x1, c_ref[:, :], wsd2[:, :], wss2[:, :],
                     aux2[:, :], (6, 7), 1)
        _pairwise_to_scratch(pf, qf, ps, qs, agg_scr, i, st2_scr)

    @pl.when(p == 2)
    def _phase2():
        out_ref[:, :] = _bn_res_relu(
            x1_scr[rows, :], agg_scr[rows, :], st2_scr, aux2[:, :])


def _make_aux(b_f, b_s, gamma, beta, wf_e, ws_e):
    return jnp.stack([b_f, b_s, gamma, beta,
                      wf_e[0], wf_e[1], ws_e[0], ws_e[1]], axis=0)


def kernel(gnn_in, centers, Wf1, bf1, Ws1, bs1, g1, be1,
           Wf2, bf2, Ws2, bs2, g2, be2, edge_index):
    del edge_index
    Wf1s, bf1s = Wf1 * 0.5, bf1 * 0.5
    Ws1s, bs1s = Ws1, bs1
    Wf2s, bf2s = Wf2 * 0.5, bf2 * 0.5
    Ws2s, bs2s = Ws2, bs2
    aux1 = _make_aux(bf1s, bs1s, g1, be1, Wf1s[2 * D:], Ws1s[2 * D:])
    aux2 = _make_aux(bf2s, bs2s, g2, be2, Wf2s[2 * D:], Ws2s[2 * D:])
    w1 = (Wf1s[:D], Wf1s[D:2 * D], Ws1s[:D], Ws1s[D:2 * D])
    w2 = (Wf2s[:D], Wf2s[D:2 * D], Ws2s[:D], Ws2s[D:2 * D])

    def full(shape):
        return pl.BlockSpec(shape, lambda p, i: tuple(0 for _ in shape))

    out = pl.pallas_call(
        _kall,
        grid=(3, GRID),
        in_specs=[pl.BlockSpec((BN, D), lambda p, i: (i, 0)),
                  pl.BlockSpec((BN, EDIM), lambda p, i: (i, 0)),
                  full((D, D)), full((D, D)), full((D, D)), full((D, D)),
                  full((8, D)),
                  full((D, D)), full((D, D)), full((D, D)), full((D, D)),
                  full((8, D))],
        out_specs=pl.BlockSpec((BN, D), lambda p, i: (i, 0)),
        out_shape=jax.ShapeDtypeStruct((N, D), jnp.float32),
        scratch_shapes=[pltpu.VMEM((N, D), jnp.float32),
                        pltpu.VMEM((N, D), jnp.float32),
                        pltpu.VMEM((8, D), jnp.float32),
                        pltpu.VMEM((8, D), jnp.float32)],
        compiler_params=pltpu.CompilerParams(
            dimension_semantics=("arbitrary", "arbitrary")),
    )(gnn_in, centers, *w1, aux1, *w2, aux2)

    return out.reshape(N_SAMPLES, N_AGENTS, D)


# speedup vs baseline: 1.0676x; 1.0676x over previous
import functools

import jax
import jax.numpy as jnp
from jax.experimental import pallas as pl
from jax.experimental.pallas import tpu as pltpu

N_SAMPLES = 512
N_AGENTS = 32
D = 128
EDIM = 2
N = N_SAMPLES * N_AGENTS

SB = 32
BN = SB * N_AGENTS
GRID = N_SAMPLES // SB
CH = 8
CHN = CH * N_AGENTS
NCHUNK = SB // CH

_LN2 = 0.6931471805599453


def _pq(xb, cb, wd, ws, aux, w_e_rows, b_row):
    ce = (cb[:, 0:1] * aux[w_e_rows[0]:w_e_rows[0] + 1, :]
          + cb[:, 1:2] * aux[w_e_rows[1]:w_e_rows[1] + 1, :])
    dot = functools.partial(jax.lax.dot_general,
                            dimension_numbers=(((1,), (0,)), ((), ())),
                            preferred_element_type=jnp.float32,
                            precision=jax.lax.Precision.HIGHEST)
    p = dot(xb, wd) + ce + aux[b_row:b_row + 1, :]
    q = dot(xb, ws) - ce
    return p, q


def _msg_scaled(v, s2):
    t = jnp.tanh(v)
    m = jnp.log(1.0 + jnp.exp(s2))
    return m + t * m


def _pairwise_chunk(pf, qf, ps, qs):
    pf3 = pf.reshape(CH, N_AGENTS, D)
    qf3 = qf.reshape(CH, N_AGENTS, D)
    ps3 = ps.reshape(CH, N_AGENTS, D)
    qs3 = qs.reshape(CH, N_AGENTS, D)
    acc = -_msg_scaled(pf + qf, ps + qs).reshape(CH, N_AGENTS, D)
    for i in range(N_AGENTS):
        acc = acc + _msg_scaled(pf3 + qf3[:, i:i + 1, :],
                                ps3 + qs3[:, i:i + 1, :])
    return acc.reshape(CHN, D) * 0.5


def _pairwise_to_scratch(pf, qf, ps, qs, agg_scr, i, st_ref):
    ssum = jnp.zeros((1, D), jnp.float32)
    ssq = jnp.zeros((1, D), jnp.float32)
    for c in range(NCHUNK):
        sl = slice(c * CHN, (c + 1) * CHN)
        aggc = _pairwise_chunk(pf[sl], qf[sl], ps[sl], qs[sl])
        agg_scr[pl.ds(i * BN + c * CHN, CHN), :] = aggc
        ssum = ssum + aggc.sum(axis=0, keepdims=True)
        ssq = ssq + (aggc * aggc).sum(axis=0, keepdims=True)
    _accum_finalize(st_ref, ssum, ssq, i)


def _accum_finalize(st_ref, ssum, ssq, i):
    @pl.when(i == 0)
    def _():
        st_ref[:, :] = jnp.zeros_like(st_ref)
    st_ref[0:1, :] = st_ref[0:1, :] + ssum
    st_ref[1:2, :] = st_ref[1:2, :] + ssq

    @pl.when(i == GRID - 1)
    def _():
        mu = st_ref[0:1, :] * (1.0 / N)
        var = st_ref[1:2, :] * (1.0 / N) - mu * mu
        st_ref[0:1, :] = mu
        st_ref[1:2, :] = jax.lax.rsqrt(var + 1e-5)


def _bn_res_relu(xb, agg, st_ref, aux):
    aggn = (agg - st_ref[0:1, :]) * st_ref[1:2, :] * aux[2:3, :] + aux[3:4, :]
    return jax.nn.relu(xb + aggn)


def _kall(x_ref, c_ref, wfd1, wfs1, wsd1, wss1, aux1,
          wfd2, wfs2, wsd2, wss2, aux2, out_ref,
          agg_scr, x1_scr, st1_scr, st2_scr):
    p = pl.program_id(0)
    i = pl.program_id(1)
    rows = pl.ds(i * BN, BN)

    @pl.when(p == 0)
    def _phase0():
        pf, qf = _pq(x_ref[:, :], c_ref[:, :], wfd1[:, :], wfs1[:, :],
                     aux1[:, :], (4, 5), 0)
        ps, qs = _pq(x_ref[:, :], c_ref[:, :], wsd1[:, :], wss1[:, :],
                     aux1[:, :], (6, 7), 1)
        _pairwise_to_scratch(pf, qf, ps, qs, agg_scr, i, st1_scr)

    @pl.when(p == 1)
    def _phase1():
        x1 = _bn_res_relu(x_ref[:, :], agg_scr[rows, :], st1_scr, aux1[:, :])
        x1_scr[rows, :] = x1
        pf, qf = _pq(x1, c_ref[:, :], wfd2[:, :], wfs2[:, :],
                     aux2[:, :], (4, 5), 0)
        ps, qs = _pq(x1, c_ref[:, :], wsd2[:, :], wss2[:, :],
                     aux2[:, :], (6, 7), 1)
        _pairwise_to_scratch(pf, qf, ps, qs, agg_scr, i, st2_scr)

    @pl.when(p == 2)
    def _phase2():
        out_ref[:, :] = _bn_res_relu(
            x1_scr[rows, :], agg_scr[rows, :], st2_scr, aux2[:, :])


def _make_aux(b_f, b_s, gamma, beta, wf_e, ws_e):
    return jnp.stack([b_f, b_s, gamma, beta,
                      wf_e[0], wf_e[1], ws_e[0], ws_e[1]], axis=0)


def kernel(gnn_in, centers, Wf1, bf1, Ws1, bs1, g1, be1,
           Wf2, bf2, Ws2, bs2, g2, be2, edge_index):
    del edge_index
    Wf1s, bf1s = Wf1 * 0.5, bf1 * 0.5
    Ws1s, bs1s = Ws1, bs1
    Wf2s, bf2s = Wf2 * 0.5, bf2 * 0.5
    Ws2s, bs2s = Ws2, bs2
    aux1 = _make_aux(bf1s, bs1s, g1, be1, Wf1s[2 * D:], Ws1s[2 * D:])
    aux2 = _make_aux(bf2s, bs2s, g2, be2, Wf2s[2 * D:], Ws2s[2 * D:])
    w1 = (Wf1s[:D], Wf1s[D:2 * D], Ws1s[:D], Ws1s[D:2 * D])
    w2 = (Wf2s[:D], Wf2s[D:2 * D], Ws2s[:D], Ws2s[D:2 * D])

    def full(shape):
        return pl.BlockSpec(shape, lambda p, i: tuple(0 for _ in shape))

    def in_map(p, i):
        return (jnp.where(p == 2, 0, i), 0)

    def out_map(p, i):
        return (jnp.where(p == 2, i, 0), 0)

    out = pl.pallas_call(
        _kall,
        grid=(3, GRID),
        in_specs=[pl.BlockSpec((BN, D), in_map),
                  pl.BlockSpec((BN, EDIM), in_map),
                  full((D, D)), full((D, D)), full((D, D)), full((D, D)),
                  full((8, D)),
                  full((D, D)), full((D, D)), full((D, D)), full((D, D)),
                  full((8, D))],
        out_specs=pl.BlockSpec((BN, D), out_map),
        out_shape=jax.ShapeDtypeStruct((N, D), jnp.float32),
        scratch_shapes=[pltpu.VMEM((N, D), jnp.float32),
                        pltpu.VMEM((N, D), jnp.float32),
                        pltpu.VMEM((8, D), jnp.float32),
                        pltpu.VMEM((8, D), jnp.float32)],
        compiler_params=pltpu.CompilerParams(
            dimension_semantics=("arbitrary", "arbitrary")),
    )(gnn_in, centers, *w1, aux1, *w2, aux2)

    return out.reshape(N_SAMPLES, N_AGENTS, D)
